# Initial kernel scaffold; baseline (speedup 1.0000x reference)
import functools

import jax
import jax.numpy as jnp
from jax import lax
from jax.experimental import pallas as pl
from jax.experimental.pallas import tpu as pltpu
from jax.experimental.pallas import tpu_sc as plsc

EMB = 32
NC, NS = 2, 16
NW = NC * NS
N = 4096 * 200
B_PER_W = N // NW
CHUNK = 1600
NCHUNK = B_PER_W // CHUNK

_mesh = plsc.VectorSubcoreMesh(core_axis_name="c", subcore_axis_name="s")


@functools.partial(
    pl.kernel,
    out_type=jax.ShapeDtypeStruct((N, EMB), jnp.float32),
    mesh=_mesh,
    scratch_types=[
        pltpu.VMEM((CHUNK,), jnp.int32),
        pltpu.VMEM((CHUNK, EMB), jnp.float32),
        pltpu.SemaphoreType.DMA,
    ],
)
def _gather_kernel(idx_hbm, table_hbm, out_hbm, idx_v, rows_v, sem):
    wid = lax.axis_index("s") * NC + lax.axis_index("c")
    base = wid * B_PER_W

    @pl.loop(0, NCHUNK)
    def _chunk(i):
        off = base + i * CHUNK
        pltpu.sync_copy(idx_hbm.at[pl.ds(off, CHUNK)], idx_v)
        pltpu.async_copy(table_hbm.at[idx_v], rows_v, sem).wait()
        pltpu.sync_copy(rows_v, out_hbm.at[pl.ds(off, CHUNK)])


def kernel(inputs, table):
    idx = inputs.reshape(-1).astype(jnp.int32)
    out = _gather_kernel(idx, table)
    return out.reshape(inputs.shape[0], inputs.shape[1], EMB)


# speedup vs baseline: 1.5603x; 1.5603x over previous
import functools

import jax
import jax.numpy as jnp
from jax import lax
from jax.experimental import pallas as pl
from jax.experimental.pallas import tpu as pltpu
from jax.experimental.pallas import tpu_sc as plsc

EMB = 32
NC, NS = 2, 16
NW = NC * NS
N = 4096 * 200
B_PER_W = N // NW
CHUNK = 1600
NCHUNK = B_PER_W // CHUNK

_mesh = plsc.VectorSubcoreMesh(core_axis_name="c", subcore_axis_name="s")


@functools.partial(
    pl.kernel,
    out_type=jax.ShapeDtypeStruct((N, EMB), jnp.float32),
    mesh=_mesh,
    scratch_types=[
        pltpu.VMEM((CHUNK,), jnp.int32),
        pltpu.VMEM((CHUNK, EMB), jnp.float32),
        pltpu.SemaphoreType.DMA,
    ],
    compiler_params=pltpu.CompilerParams(use_tc_tiling_on_sc=False),
)
def _gather_kernel(idx_hbm, table_hbm, out_hbm, idx_v, rows_v, sem):
    wid = lax.axis_index("s") * NC + lax.axis_index("c")
    base = wid * B_PER_W

    @pl.loop(0, NCHUNK)
    def _chunk(i):
        off = base + i * CHUNK
        pltpu.sync_copy(idx_hbm.at[pl.ds(off, CHUNK)], idx_v)
        pltpu.async_copy(table_hbm.at[idx_v], rows_v, sem).wait()
        pltpu.sync_copy(rows_v, out_hbm.at[pl.ds(off, CHUNK)])


def kernel(inputs, table):
    idx = inputs.reshape(-1).astype(jnp.int32)
    out = _gather_kernel(idx, table)
    return out.reshape(inputs.shape[0], inputs.shape[1], EMB)


# speedup vs baseline: 1.5818x; 1.0138x over previous
import functools

import jax
import jax.numpy as jnp
from jax import lax
from jax.experimental import pallas as pl
from jax.experimental.pallas import tpu as pltpu
from jax.experimental.pallas import tpu_sc as plsc

EMB = 32
NC, NS = 2, 16
NW = NC * NS
N = 4096 * 200
B_PER_W = N // NW
NBUF = 4
CHUNK = 800
NCHUNK = B_PER_W // CHUNK

_mesh = plsc.VectorSubcoreMesh(core_axis_name="c", subcore_axis_name="s")


@functools.partial(
    pl.kernel,
    out_type=jax.ShapeDtypeStruct((N, EMB), jnp.float32),
    mesh=_mesh,
    scratch_types=(
        [
            pltpu.VMEM((B_PER_W,), jnp.int32),
            pltpu.VMEM((NBUF, CHUNK, EMB), jnp.float32),
        ]
        + [pltpu.SemaphoreType.DMA] * (2 * NBUF)
    ),
    compiler_params=pltpu.CompilerParams(use_tc_tiling_on_sc=False),
)
def _gather_kernel(idx_hbm, table_hbm, out_hbm, idx_all, rows, *sems):
    gsem = sems[:NBUF]
    wsem = sems[NBUF:]
    wid = lax.axis_index("s") * NC + lax.axis_index("c")
    base = wid * B_PER_W

    pltpu.sync_copy(idx_hbm.at[pl.ds(base, B_PER_W)], idx_all)

    def gather(c, b):
        return pltpu.make_async_copy(
            table_hbm.at[idx_all.at[pl.ds(c * CHUNK, CHUNK)]],
            rows.at[b],
            gsem[b],
        )

    def writeback(c, b):
        return pltpu.make_async_copy(
            rows.at[b],
            out_hbm.at[pl.ds(base + c * CHUNK, CHUNK)],
            wsem[b],
        )

    for b in range(NBUF):
        gather(b, b).start()

    @pl.loop(0, NCHUNK - NBUF, step=NBUF)
    def _outer(i):
        for b in range(NBUF):
            c = i + b
            gather(c, b).wait()
            writeback(c, b).start()
            writeback(c, b).wait()
            gather(c + NBUF, b).start()

    for b in range(NBUF):
        c = NCHUNK - NBUF + b
        gather(c, b).wait()
        writeback(c, b).start()
    for b in range(NBUF):
        writeback(NCHUNK - NBUF + b, b).wait()


def kernel(inputs, table):
    idx = inputs.reshape(-1).astype(jnp.int32)
    out = _gather_kernel(idx, table)
    return out.reshape(inputs.shape[0], inputs.shape[1], EMB)


# speedup vs baseline: 1.5860x; 1.0026x over previous
import functools

import jax
import jax.numpy as jnp
from jax import lax
from jax.experimental import pallas as pl
from jax.experimental.pallas import tpu as pltpu
from jax.experimental.pallas import tpu_sc as plsc

VOCAB = 1000000
EMB = 32
B = 4096
L = 200
NC, NS = 2, 16
NW = NC * NS
B_PER_W = B // NW
NBUF = 8

_mesh = plsc.VectorSubcoreMesh(core_axis_name="c", subcore_axis_name="s")


@functools.partial(
    pl.kernel,
    out_type=jax.ShapeDtypeStruct((B, L, EMB), jnp.float32),
    mesh=_mesh,
    scratch_types=(
        [
            pltpu.VMEM((B_PER_W, L), jnp.int32),
            pltpu.VMEM((NBUF, L, EMB), jnp.float32),
        ]
        + [pltpu.SemaphoreType.DMA] * (2 * NBUF)
    ),
    compiler_params=pltpu.CompilerParams(use_tc_tiling_on_sc=False),
)
def _gather_kernel(idx_hbm, table_hbm, out_hbm, idx_all, rows, *sems):
    gsem = sems[:NBUF]
    wsem = sems[NBUF:]
    wid = lax.axis_index("s") * NC + lax.axis_index("c")
    base = wid * B_PER_W

    pltpu.sync_copy(idx_hbm.at[pl.ds(base, B_PER_W)], idx_all)

    def gather(c, b):
        return pltpu.make_async_copy(
            table_hbm.at[idx_all.at[c]],
            rows.at[b],
            gsem[b],
        )

    def writeback(c, b):
        return pltpu.make_async_copy(
            rows.at[b],
            out_hbm.at[base + c],
            wsem[b],
        )

    for b in range(NBUF):
        gather(b, b).start()

    @pl.loop(0, B_PER_W - NBUF, step=NBUF)
    def _outer(i):
        for b in range(NBUF):
            c = i + b
            gather(c, b).wait()
            writeback(c, b).start()
            writeback(c, b).wait()
            gather(c + NBUF, b).start()

    for b in range(NBUF):
        c = B_PER_W - NBUF + b
        gather(c, b).wait()
        writeback(c, b).start()
    for b in range(NBUF):
        writeback(B_PER_W - NBUF + b, b).wait()


def kernel(inputs, table):
    return _gather_kernel(inputs.astype(jnp.int32), table)
